# baseline (device time: 37782 ns/iter reference)
import jax
import jax.numpy as jnp
from jax import lax
from jax.experimental import pallas as pl
from jax.experimental.pallas import tpu as pltpu

N_Z = 4
M = 2048
D = 512
CHUNK = M // N_Z


def kernel(partial, gamma):
    gamma2 = gamma.reshape(1, D)

    def body(p_ref, g_ref, out_ref, send_buf, recv_buf, send_sems, recv_sems):
        my_x = lax.axis_index("x")
        my_y = lax.axis_index("y")
        my_z = lax.axis_index("z")
        right = (my_z + 1) % N_Z

        def local_chunk(c):
            return p_ref[0, pl.ds(c * CHUNK, CHUNK), :].astype(jnp.bfloat16)

        send_buf[0] = local_chunk((my_z + N_Z - 1) % N_Z)

        acc = None
        for s in range(N_Z - 1):
            rdma = pltpu.make_async_remote_copy(
                src_ref=send_buf.at[s],
                dst_ref=recv_buf.at[s],
                send_sem=send_sems.at[s],
                recv_sem=recv_sems.at[s],
                device_id=(my_x, my_y, right),
                device_id_type=pl.DeviceIdType.MESH,
            )
            rdma.start()
            rdma.wait()
            acc = recv_buf[s] + local_chunk((my_z + 2 - s) % N_Z)
            if s < N_Z - 2:
                send_buf[s + 1] = acc

        y = acc.astype(jnp.float32)
        rms = jnp.sqrt(jnp.mean(y * y, axis=-1, keepdims=True) + 1e-6)
        out_ref[...] = y / rms * g_ref[...]

    return pl.pallas_call(
        body,
        out_shape=jax.ShapeDtypeStruct((CHUNK, D), jnp.float32),
        in_specs=[
            pl.BlockSpec(memory_space=pltpu.VMEM),
            pl.BlockSpec(memory_space=pltpu.VMEM),
        ],
        out_specs=pl.BlockSpec(memory_space=pltpu.VMEM),
        scratch_shapes=[
            pltpu.VMEM((N_Z - 1, CHUNK, D), jnp.bfloat16),
            pltpu.VMEM((N_Z - 1, CHUNK, D), jnp.bfloat16),
            pltpu.SemaphoreType.DMA((N_Z - 1,)),
            pltpu.SemaphoreType.DMA((N_Z - 1,)),
        ],
    )(partial, gamma2)
